# Initial kernel scaffold; baseline (speedup 1.0000x reference)
import jax
import jax.numpy as jnp
from jax.experimental import pallas as pl


def kernel(hidden_states, Wg, W1, W2, W3):
    raise NotImplementedError("write your pallas kernel here")


# speedup vs baseline: 1.1783x; 1.1783x over previous
import functools

import jax
import jax.numpy as jnp
from jax.experimental import pallas as pl
from jax.experimental.pallas import tpu as pltpu

E = 8
D = 1024
F = 4096
T = 2048
BM = 128
S = T * 2 + E * BM
MB = S // BM
BF = 1024
NF = F // BF
TCHUNK = 128
NCHUNK = T // TCHUNK


def _gate_body(x_ref, wg_ref, e_ref, w_ref, r_ref, cnt_ref, carry_ref):
    c = pl.program_id(0)

    @pl.when(c == 0)
    def _init():
        carry_ref[...] = jnp.zeros_like(carry_ref)

    x = x_ref[...]
    wg = wg_ref[...]
    logits = jax.lax.dot_general(
        x, wg, (((1,), (1,)), ((), ())),
        preferred_element_type=jnp.float32)
    m = jnp.max(logits, axis=-1, keepdims=True)
    p = jnp.exp(logits - m)
    probs = p / jnp.sum(p, axis=-1, keepdims=True)

    idx = jax.lax.broadcasted_iota(jnp.int32, (TCHUNK, E), 1)
    m0 = jnp.max(logits, axis=-1, keepdims=True)
    e0 = jnp.min(jnp.where(logits >= m0, idx, E), axis=-1, keepdims=True)
    oh0 = (idx == e0)
    masked = jnp.where(oh0, -jnp.inf, logits)
    m1 = jnp.max(masked, axis=-1, keepdims=True)
    e1 = jnp.min(jnp.where(masked >= m1, idx, E), axis=-1, keepdims=True)
    oh1 = (idx == e1)

    oh0f = oh0.astype(jnp.float32)
    oh1f = oh1.astype(jnp.float32)
    p0 = jnp.sum(probs * oh0f, axis=-1, keepdims=True)
    p1 = jnp.sum(probs * oh1f, axis=-1, keepdims=True)
    tot = p0 + p1
    w0 = p0 / tot
    w1 = p1 / tot

    ii = jax.lax.broadcasted_iota(jnp.int32, (TCHUNK, TCHUNK), 0)
    jj = jax.lax.broadcasted_iota(jnp.int32, (TCHUNK, TCHUNK), 1)
    tril = (jj < ii).astype(jnp.float32)
    carry = carry_ref[...]
    r0 = jax.lax.dot_general(tril, oh0f, (((1,), (0,)), ((), ())),
                             preferred_element_type=jnp.float32)
    rank0 = jnp.sum((carry + r0) * oh0f, axis=-1, keepdims=True)
    mid = carry + jnp.sum(oh0f, axis=0, keepdims=True)
    r1 = jax.lax.dot_general(tril, oh1f, (((1,), (0,)), ((), ())),
                             preferred_element_type=jnp.float32)
    rank1 = jnp.sum((mid + r1) * oh1f, axis=-1, keepdims=True)
    new_carry = mid + jnp.sum(oh1f, axis=0, keepdims=True)
    carry_ref[...] = new_carry

    e_ref[...] = jnp.concatenate([e0, e1], axis=1)
    w_ref[...] = jnp.concatenate([w0, w1], axis=1)
    r_ref[...] = jnp.concatenate([rank0, rank1], axis=1)
    cnt_ref[...] = jnp.broadcast_to(new_carry, (8, E))


def _gate_call(x, wg, *, interpret=False):
    return pl.pallas_call(
        _gate_body,
        grid=(NCHUNK,),
        in_specs=[
            pl.BlockSpec((TCHUNK, D), lambda c: (c, 0)),
            pl.BlockSpec((E, D), lambda c: (0, 0)),
        ],
        out_specs=[
            pl.BlockSpec((TCHUNK, 2), lambda c: (c, 0)),
            pl.BlockSpec((TCHUNK, 2), lambda c: (c, 0)),
            pl.BlockSpec((TCHUNK, 2), lambda c: (c, 0)),
            pl.BlockSpec((8, E), lambda c: (0, 0)),
        ],
        out_shape=[
            jax.ShapeDtypeStruct((T, 2), jnp.int32),
            jax.ShapeDtypeStruct((T, 2), jnp.float32),
            jax.ShapeDtypeStruct((T, 2), jnp.float32),
            jax.ShapeDtypeStruct((8, E), jnp.float32),
        ],
        scratch_shapes=[pltpu.VMEM((1, E), jnp.float32)],
        interpret=interpret,
    )(x, wg)


def _dispatch_body(s0_ref, s1_ref, x_ref, xs_ref):
    b = pl.program_id(0)
    sid = b * BM + jax.lax.broadcasted_iota(jnp.int32, (BM, T), 0)
    sel = ((sid == s0_ref[...]) | (sid == s1_ref[...])).astype(jnp.float32)
    xs_ref[...] = jax.lax.dot_general(
        sel, x_ref[...], (((1,), (0,)), ((), ())),
        preferred_element_type=jnp.float32)


def _dispatch_call(slot0, slot1, x, *, interpret=False):
    return pl.pallas_call(
        _dispatch_body,
        grid=(MB,),
        in_specs=[
            pl.BlockSpec((1, T), lambda b: (0, 0)),
            pl.BlockSpec((1, T), lambda b: (0, 0)),
            pl.BlockSpec((T, D), lambda b: (0, 0)),
        ],
        out_specs=pl.BlockSpec((BM, D), lambda b: (b, 0)),
        out_shape=jax.ShapeDtypeStruct((S, D), jnp.float32),
        interpret=interpret,
    )(slot0, slot1, x)


def _ffn1_body(be_ref, xs_ref, w1_ref, w3_ref, h_ref):
    x = xs_ref[...]
    a = jax.lax.dot_general(x, w1_ref[0], (((1,), (1,)), ((), ())),
                            preferred_element_type=jnp.float32)
    g = jax.lax.dot_general(x, w3_ref[0], (((1,), (1,)), ((), ())),
                            preferred_element_type=jnp.float32)
    h_ref[...] = (a * jax.lax.logistic(a)) * g


def _ffn1_call(block_expert, xs, W1, W3, *, interpret=False):
    grid_spec = pltpu.PrefetchScalarGridSpec(
        num_scalar_prefetch=1,
        grid=(NF, MB),
        in_specs=[
            pl.BlockSpec((BM, D), lambda f, m, be: (m, 0)),
            pl.BlockSpec((1, BF, D), lambda f, m, be: (be[m], f, 0)),
            pl.BlockSpec((1, BF, D), lambda f, m, be: (be[m], f, 0)),
        ],
        out_specs=pl.BlockSpec((BM, BF), lambda f, m, be: (m, f)),
    )
    return pl.pallas_call(
        _ffn1_body,
        grid_spec=grid_spec,
        out_shape=jax.ShapeDtypeStruct((S, F), jnp.float32),
        interpret=interpret,
    )(block_expert, xs, W1, W3)


def _ffn2_body(be_ref, h_ref, w2_ref, y_ref):
    y_ref[...] = jax.lax.dot_general(
        h_ref[...], w2_ref[0], (((1,), (1,)), ((), ())),
        preferred_element_type=jnp.float32)


def _ffn2_call(block_expert, h, W2, *, interpret=False):
    grid_spec = pltpu.PrefetchScalarGridSpec(
        num_scalar_prefetch=1,
        grid=(MB,),
        in_specs=[
            pl.BlockSpec((BM, F), lambda m, be: (m, 0)),
            pl.BlockSpec((1, D, F), lambda m, be: (be[m], 0, 0)),
        ],
        out_specs=pl.BlockSpec((BM, D), lambda m, be: (m, 0)),
    )
    return pl.pallas_call(
        _ffn2_body,
        grid_spec=grid_spec,
        out_shape=jax.ShapeDtypeStruct((S, D), jnp.float32),
        interpret=interpret,
    )(block_expert, h, W2)


def _combine_body(slot_ref, w_ref, y_ref, out_ref):
    s = slot_ref[...]
    w = w_ref[...]
    sid = jax.lax.broadcasted_iota(jnp.int32, (TCHUNK, S), 1)
    comb = (jnp.where(sid == s[:, 0:1], w[:, 0:1], 0.0)
            + jnp.where(sid == s[:, 1:2], w[:, 1:2], 0.0))
    out_ref[...] = jax.lax.dot_general(
        comb, y_ref[...], (((1,), (0,)), ((), ())),
        preferred_element_type=jnp.float32)


def _combine_call(slot, w, y, *, interpret=False):
    return pl.pallas_call(
        _combine_body,
        grid=(NCHUNK,),
        in_specs=[
            pl.BlockSpec((TCHUNK, 2), lambda c: (c, 0)),
            pl.BlockSpec((TCHUNK, 2), lambda c: (c, 0)),
            pl.BlockSpec((S, D), lambda c: (0, 0)),
        ],
        out_specs=pl.BlockSpec((TCHUNK, D), lambda c: (c, 0)),
        out_shape=jax.ShapeDtypeStruct((T, D), jnp.float32),
        interpret=interpret,
    )(slot, w, y)


def _moe_impl(hidden_states, Wg, W1, W2, W3, *, interpret=False):
    x = hidden_states.reshape(T, D)
    e, w, r, cnt = _gate_call(x, Wg, interpret=interpret)
    counts = cnt[0].astype(jnp.int32)
    pblocks = (counts + BM - 1) // BM
    starts = jnp.concatenate(
        [jnp.zeros((1,), jnp.int32), jnp.cumsum(pblocks)[:-1]])
    off = starts * BM
    slot = jnp.take(off, e, axis=0) + r.astype(jnp.int32)
    bidx = jnp.arange(MB, dtype=jnp.int32)
    block_expert = (jnp.sum((bidx[:, None] >= starts[None, :]).astype(jnp.int32),
                            axis=1) - 1).astype(jnp.int32)
    slot0 = slot[:, 0].reshape(1, T)
    slot1 = slot[:, 1].reshape(1, T)
    xs = _dispatch_call(slot0, slot1, x, interpret=interpret)
    h = _ffn1_call(block_expert, xs, W1, W3, interpret=interpret)
    y = _ffn2_call(block_expert, h, W2, interpret=interpret)
    out = _combine_call(slot, w, y, interpret=interpret)
    return out.reshape(hidden_states.shape)


def kernel(hidden_states, Wg, W1, W2, W3):
    return _moe_impl(hidden_states, Wg, W1, W2, W3)


# speedup vs baseline: 1.1799x; 1.0013x over previous
import functools

import jax
import jax.numpy as jnp
from jax.experimental import pallas as pl
from jax.experimental.pallas import tpu as pltpu

E = 8
D = 1024
F = 4096
T = 2048
BM = 128
S = T * 2 + E * BM
MB = S // BM
BF = 1024
NF = F // BF
TCHUNK = 128
NCHUNK = T // TCHUNK


def _gate_body(x_ref, wg_ref, e_ref, w_ref, r_ref, cnt_ref, carry_ref):
    c = pl.program_id(0)

    @pl.when(c == 0)
    def _init():
        carry_ref[...] = jnp.zeros_like(carry_ref)

    x = x_ref[...]
    wg = wg_ref[...]
    logits = jax.lax.dot_general(
        x, wg, (((1,), (1,)), ((), ())),
        preferred_element_type=jnp.float32)
    m = jnp.max(logits, axis=-1, keepdims=True)
    p = jnp.exp(logits - m)
    probs = p / jnp.sum(p, axis=-1, keepdims=True)

    idx = jax.lax.broadcasted_iota(jnp.int32, (TCHUNK, E), 1)
    m0 = jnp.max(logits, axis=-1, keepdims=True)
    e0 = jnp.min(jnp.where(logits >= m0, idx, E), axis=-1, keepdims=True)
    oh0 = (idx == e0)
    masked = jnp.where(oh0, -jnp.inf, logits)
    m1 = jnp.max(masked, axis=-1, keepdims=True)
    e1 = jnp.min(jnp.where(masked >= m1, idx, E), axis=-1, keepdims=True)
    oh1 = (idx == e1)

    oh0f = oh0.astype(jnp.float32)
    oh1f = oh1.astype(jnp.float32)
    p0 = jnp.sum(probs * oh0f, axis=-1, keepdims=True)
    p1 = jnp.sum(probs * oh1f, axis=-1, keepdims=True)
    tot = p0 + p1
    w0 = p0 / tot
    w1 = p1 / tot

    ii = jax.lax.broadcasted_iota(jnp.int32, (TCHUNK, TCHUNK), 0)
    jj = jax.lax.broadcasted_iota(jnp.int32, (TCHUNK, TCHUNK), 1)
    tril = (jj < ii).astype(jnp.float32)
    carry = carry_ref[...]
    r0 = jax.lax.dot_general(tril, oh0f, (((1,), (0,)), ((), ())),
                             preferred_element_type=jnp.float32)
    rank0 = jnp.sum((carry + r0) * oh0f, axis=-1, keepdims=True)
    mid = carry + jnp.sum(oh0f, axis=0, keepdims=True)
    r1 = jax.lax.dot_general(tril, oh1f, (((1,), (0,)), ((), ())),
                             preferred_element_type=jnp.float32)
    rank1 = jnp.sum((mid + r1) * oh1f, axis=-1, keepdims=True)
    new_carry = mid + jnp.sum(oh1f, axis=0, keepdims=True)
    carry_ref[...] = new_carry

    e_ref[...] = jnp.concatenate([e0, e1], axis=1)
    w_ref[...] = jnp.concatenate([w0, w1], axis=1)
    r_ref[...] = jnp.concatenate([rank0, rank1], axis=1)
    cnt_ref[...] = jnp.broadcast_to(new_carry, (8, E))


def _gate_call(x, wg, *, interpret=False):
    return pl.pallas_call(
        _gate_body,
        grid=(NCHUNK,),
        in_specs=[
            pl.BlockSpec((TCHUNK, D), lambda c: (c, 0)),
            pl.BlockSpec((E, D), lambda c: (0, 0)),
        ],
        out_specs=[
            pl.BlockSpec((TCHUNK, 2), lambda c: (c, 0)),
            pl.BlockSpec((TCHUNK, 2), lambda c: (c, 0)),
            pl.BlockSpec((TCHUNK, 2), lambda c: (c, 0)),
            pl.BlockSpec((8, E), lambda c: (0, 0)),
        ],
        out_shape=[
            jax.ShapeDtypeStruct((T, 2), jnp.int32),
            jax.ShapeDtypeStruct((T, 2), jnp.float32),
            jax.ShapeDtypeStruct((T, 2), jnp.float32),
            jax.ShapeDtypeStruct((8, E), jnp.float32),
        ],
        scratch_shapes=[pltpu.VMEM((1, E), jnp.float32)],
        interpret=interpret,
    )(x, wg)


def _dispatch_body(s0_ref, s1_ref, x_ref, xs_ref):
    b = pl.program_id(0)
    sid = b * BM + jax.lax.broadcasted_iota(jnp.int32, (BM, T), 0)
    sel = ((sid == s0_ref[...]) | (sid == s1_ref[...])).astype(jnp.float32)
    xs_ref[...] = jax.lax.dot_general(
        sel, x_ref[...], (((1,), (0,)), ((), ())),
        preferred_element_type=jnp.float32).astype(jnp.bfloat16)


def _dispatch_call(slot0, slot1, x, *, interpret=False):
    return pl.pallas_call(
        _dispatch_body,
        grid=(MB,),
        in_specs=[
            pl.BlockSpec((1, T), lambda b: (0, 0)),
            pl.BlockSpec((1, T), lambda b: (0, 0)),
            pl.BlockSpec((T, D), lambda b: (0, 0)),
        ],
        out_specs=pl.BlockSpec((BM, D), lambda b: (b, 0)),
        out_shape=jax.ShapeDtypeStruct((S, D), jnp.bfloat16),
        interpret=interpret,
    )(slot0, slot1, x)


HBF = BF // 2


def _ffn1_body(be_ref, xs_ref, w1a_ref, w1b_ref, w3a_ref, w3b_ref, h_ref):
    x = xs_ref[...].astype(jnp.float32)
    for half, w1_ref, w3_ref in ((0, w1a_ref, w3a_ref), (1, w1b_ref, w3b_ref)):
        a = jax.lax.dot_general(x, w1_ref[0], (((1,), (1,)), ((), ())),
                                preferred_element_type=jnp.float32)
        g = jax.lax.dot_general(x, w3_ref[0], (((1,), (1,)), ((), ())),
                                preferred_element_type=jnp.float32)
        h_ref[:, half * HBF:(half + 1) * HBF] = (
            (a * jax.lax.logistic(a)) * g).astype(jnp.bfloat16)


def _ffn1_call(block_expert, xs, W1, W3, *, interpret=False):
    grid_spec = pltpu.PrefetchScalarGridSpec(
        num_scalar_prefetch=1,
        grid=(NF, MB),
        in_specs=[
            pl.BlockSpec((BM, D), lambda f, m, be: (m, 0)),
            pl.BlockSpec((1, HBF, D), lambda f, m, be: (be[m], 2 * f, 0)),
            pl.BlockSpec((1, HBF, D), lambda f, m, be: (be[m], 2 * f + 1, 0)),
            pl.BlockSpec((1, HBF, D), lambda f, m, be: (be[m], 2 * f, 0)),
            pl.BlockSpec((1, HBF, D), lambda f, m, be: (be[m], 2 * f + 1, 0)),
        ],
        out_specs=pl.BlockSpec((BM, BF), lambda f, m, be: (m, f)),
    )
    return pl.pallas_call(
        _ffn1_body,
        grid_spec=grid_spec,
        out_shape=jax.ShapeDtypeStruct((S, F), jnp.bfloat16),
        interpret=interpret,
    )(block_expert, xs, W1, W1, W3, W3)


QD = D // 4


def _ffn2_body(be_ref, h_ref, w2a_ref, w2b_ref, w2c_ref, w2d_ref, y_ref):
    h = h_ref[...].astype(jnp.float32)
    for q, wr in enumerate((w2a_ref, w2b_ref, w2c_ref, w2d_ref)):
        y_ref[:, q * QD:(q + 1) * QD] = jax.lax.dot_general(
            h, wr[0], (((1,), (1,)), ((), ())),
            preferred_element_type=jnp.float32)


def _ffn2_call(block_expert, h, W2, *, interpret=False):
    grid_spec = pltpu.PrefetchScalarGridSpec(
        num_scalar_prefetch=1,
        grid=(MB,),
        in_specs=[
            pl.BlockSpec((BM, F), lambda m, be: (m, 0)),
            pl.BlockSpec((1, QD, F), lambda m, be: (be[m], 0, 0)),
            pl.BlockSpec((1, QD, F), lambda m, be: (be[m], 1, 0)),
            pl.BlockSpec((1, QD, F), lambda m, be: (be[m], 2, 0)),
            pl.BlockSpec((1, QD, F), lambda m, be: (be[m], 3, 0)),
        ],
        out_specs=pl.BlockSpec((BM, D), lambda m, be: (m, 0)),
    )
    return pl.pallas_call(
        _ffn2_body,
        grid_spec=grid_spec,
        out_shape=jax.ShapeDtypeStruct((S, D), jnp.float32),
        interpret=interpret,
    )(block_expert, h, W2, W2, W2, W2)


def _combine_body(slot_ref, w_ref, y_ref, out_ref):
    s = slot_ref[...]
    w = w_ref[...]
    sid = jax.lax.broadcasted_iota(jnp.int32, (TCHUNK, S), 1)
    comb = (jnp.where(sid == s[:, 0:1], w[:, 0:1], 0.0)
            + jnp.where(sid == s[:, 1:2], w[:, 1:2], 0.0))
    out_ref[...] = jax.lax.dot_general(
        comb, y_ref[...], (((1,), (0,)), ((), ())),
        preferred_element_type=jnp.float32)


def _combine_call(slot, w, y, *, interpret=False):
    return pl.pallas_call(
        _combine_body,
        grid=(NCHUNK,),
        in_specs=[
            pl.BlockSpec((TCHUNK, 2), lambda c: (c, 0)),
            pl.BlockSpec((TCHUNK, 2), lambda c: (c, 0)),
            pl.BlockSpec((S, D), lambda c: (0, 0)),
        ],
        out_specs=pl.BlockSpec((TCHUNK, D), lambda c: (c, 0)),
        out_shape=jax.ShapeDtypeStruct((T, D), jnp.float32),
        interpret=interpret,
    )(slot, w, y)


def _moe_impl(hidden_states, Wg, W1, W2, W3, *, interpret=False):
    x = hidden_states.reshape(T, D)
    e, w, r, cnt = _gate_call(x, Wg, interpret=interpret)
    counts = cnt[0].astype(jnp.int32)
    pblocks = (counts + BM - 1) // BM
    starts = jnp.concatenate(
        [jnp.zeros((1,), jnp.int32), jnp.cumsum(pblocks)[:-1]])
    off = starts * BM
    slot = jnp.take(off, e, axis=0) + r.astype(jnp.int32)
    bidx = jnp.arange(MB, dtype=jnp.int32)
    block_expert = (jnp.sum((bidx[:, None] >= starts[None, :]).astype(jnp.int32),
                            axis=1) - 1).astype(jnp.int32)
    slot0 = slot[:, 0].reshape(1, T)
    slot1 = slot[:, 1].reshape(1, T)
    xs = _dispatch_call(slot0, slot1, x, interpret=interpret)
    h = _ffn1_call(block_expert, xs, W1, W3, interpret=interpret)
    y = _ffn2_call(block_expert, h, W2, interpret=interpret)
    out = _combine_call(slot, w, y, interpret=interpret)
    return out.reshape(hidden_states.shape)


def kernel(hidden_states, Wg, W1, W2, W3):
    return _moe_impl(hidden_states, Wg, W1, W2, W3)


# speedup vs baseline: 1.5611x; 1.3231x over previous
import functools

import jax
import jax.numpy as jnp
from jax.experimental import pallas as pl
from jax.experimental.pallas import tpu as pltpu

E = 8
D = 1024
F = 4096
T = 2048
BM = 128
S = T * 2 + E * BM
MB = S // BM
BF = 1024
NF = F // BF
TCHUNK = 128
NCHUNK = T // TCHUNK


def _gate_body(x_ref, wg_ref, e_ref, w_ref, r_ref, cnt_ref, carry_ref):
    c = pl.program_id(0)

    @pl.when(c == 0)
    def _init():
        carry_ref[...] = jnp.zeros_like(carry_ref)

    x = x_ref[...]
    wg = wg_ref[...]
    logits = jax.lax.dot_general(
        x, wg, (((1,), (1,)), ((), ())),
        preferred_element_type=jnp.float32)
    m = jnp.max(logits, axis=-1, keepdims=True)
    p = jnp.exp(logits - m)
    probs = p / jnp.sum(p, axis=-1, keepdims=True)

    idx = jax.lax.broadcasted_iota(jnp.int32, (TCHUNK, E), 1)
    m0 = jnp.max(logits, axis=-1, keepdims=True)
    e0 = jnp.min(jnp.where(logits >= m0, idx, E), axis=-1, keepdims=True)
    oh0 = (idx == e0)
    masked = jnp.where(oh0, -jnp.inf, logits)
    m1 = jnp.max(masked, axis=-1, keepdims=True)
    e1 = jnp.min(jnp.where(masked >= m1, idx, E), axis=-1, keepdims=True)
    oh1 = (idx == e1)

    oh0f = oh0.astype(jnp.float32)
    oh1f = oh1.astype(jnp.float32)
    p0 = jnp.sum(probs * oh0f, axis=-1, keepdims=True)
    p1 = jnp.sum(probs * oh1f, axis=-1, keepdims=True)
    tot = p0 + p1
    w0 = p0 / tot
    w1 = p1 / tot

    ii = jax.lax.broadcasted_iota(jnp.int32, (TCHUNK, TCHUNK), 0)
    jj = jax.lax.broadcasted_iota(jnp.int32, (TCHUNK, TCHUNK), 1)
    tril = (jj < ii).astype(jnp.float32)
    carry = carry_ref[...]
    r0 = jax.lax.dot_general(tril, oh0f, (((1,), (0,)), ((), ())),
                             preferred_element_type=jnp.float32)
    rank0 = jnp.sum((carry + r0) * oh0f, axis=-1, keepdims=True)
    mid = carry + jnp.sum(oh0f, axis=0, keepdims=True)
    r1 = jax.lax.dot_general(tril, oh1f, (((1,), (0,)), ((), ())),
                             preferred_element_type=jnp.float32)
    rank1 = jnp.sum((mid + r1) * oh1f, axis=-1, keepdims=True)
    new_carry = mid + jnp.sum(oh1f, axis=0, keepdims=True)
    carry_ref[...] = new_carry

    e_ref[...] = jnp.concatenate([e0, e1], axis=1)
    w_ref[...] = jnp.concatenate([w0, w1], axis=1)
    r_ref[...] = jnp.concatenate([rank0, rank1], axis=1)
    cnt_ref[...] = jnp.broadcast_to(new_carry, (8, E))


def _gate_call(x, wg, *, interpret=False):
    return pl.pallas_call(
        _gate_body,
        grid=(NCHUNK,),
        in_specs=[
            pl.BlockSpec((TCHUNK, D), lambda c: (c, 0)),
            pl.BlockSpec((E, D), lambda c: (0, 0)),
        ],
        out_specs=[
            pl.BlockSpec((TCHUNK, 2), lambda c: (c, 0)),
            pl.BlockSpec((TCHUNK, 2), lambda c: (c, 0)),
            pl.BlockSpec((TCHUNK, 2), lambda c: (c, 0)),
            pl.BlockSpec((8, E), lambda c: (0, 0)),
        ],
        out_shape=[
            jax.ShapeDtypeStruct((T, 2), jnp.int32),
            jax.ShapeDtypeStruct((T, 2), jnp.float32),
            jax.ShapeDtypeStruct((T, 2), jnp.float32),
            jax.ShapeDtypeStruct((8, E), jnp.float32),
        ],
        scratch_shapes=[pltpu.VMEM((1, E), jnp.float32)],
        interpret=interpret,
    )(x, wg)


def _dispatch_body(s0_ref, s1_ref, x_ref, xs_ref):
    b = pl.program_id(0)
    sid = b * BM + jax.lax.broadcasted_iota(jnp.int32, (BM, T), 0)
    sel = ((sid == s0_ref[...]) | (sid == s1_ref[...])).astype(jnp.float32)
    xs_ref[...] = jax.lax.dot_general(
        sel, x_ref[...], (((1,), (0,)), ((), ())),
        preferred_element_type=jnp.float32).astype(jnp.bfloat16)


def _dispatch_call(slot0, slot1, x, *, interpret=False):
    return pl.pallas_call(
        _dispatch_body,
        grid=(MB,),
        in_specs=[
            pl.BlockSpec((1, T), lambda b: (0, 0)),
            pl.BlockSpec((1, T), lambda b: (0, 0)),
            pl.BlockSpec((T, D), lambda b: (0, 0)),
        ],
        out_specs=pl.BlockSpec((BM, D), lambda b: (b, 0)),
        out_shape=jax.ShapeDtypeStruct((S, D), jnp.bfloat16),
        interpret=interpret,
    )(slot0, slot1, x)


def _ffn_body(meta_ref, w1_ref, w3_ref, w2_ref, xs_hbm, y_hbm,
              xs_v, y_v, sem_in, sem_out):
    e = pl.program_id(0)
    f = pl.program_id(1)

    @pl.when((e == 0) & (f == 0))
    def _load_xs():
        cp = pltpu.make_async_copy(xs_hbm, xs_v, sem_in)
        cp.start()
        y_v[...] = jnp.zeros_like(y_v)
        cp.wait()

    nb = meta_ref[e]
    base = meta_ref[E + e]

    def blk(b, carry):
        r0 = (base + b) * BM
        x = xs_v[pl.ds(r0, BM), :].astype(jnp.float32)
        a = jax.lax.dot_general(x, w1_ref[0], (((1,), (1,)), ((), ())),
                                preferred_element_type=jnp.float32)
        g = jax.lax.dot_general(x, w3_ref[0], (((1,), (1,)), ((), ())),
                                preferred_element_type=jnp.float32)
        h = (a * jax.lax.logistic(a)) * g
        yp = jax.lax.dot_general(h, w2_ref[0], (((1,), (1,)), ((), ())),
                                 preferred_element_type=jnp.float32)
        prev = y_v[pl.ds(r0, BM), :]
        y_v[pl.ds(r0, BM), :] = jnp.where(f == 0, yp, prev + yp)
        return carry

    jax.lax.fori_loop(0, nb, blk, 0)

    @pl.when((e == E - 1) & (f == NF - 1))
    def _store_y():
        cp = pltpu.make_async_copy(y_v, y_hbm, sem_out)
        cp.start()
        cp.wait()


def _ffn_call(meta, xs, W1, W3, W2, *, interpret=False):
    grid_spec = pltpu.PrefetchScalarGridSpec(
        num_scalar_prefetch=1,
        grid=(E, NF),
        in_specs=[
            pl.BlockSpec((1, BF, D), lambda e, f, meta: (e, f, 0)),
            pl.BlockSpec((1, BF, D), lambda e, f, meta: (e, f, 0)),
            pl.BlockSpec((1, D, BF), lambda e, f, meta: (e, 0, f)),
            pl.BlockSpec(memory_space=pl.ANY),
        ],
        out_specs=pl.BlockSpec(memory_space=pl.ANY),
        scratch_shapes=[
            pltpu.VMEM((S, D), jnp.bfloat16),
            pltpu.VMEM((S, D), jnp.float32),
            pltpu.SemaphoreType.DMA,
            pltpu.SemaphoreType.DMA,
        ],
    )
    return pl.pallas_call(
        _ffn_body,
        grid_spec=grid_spec,
        out_shape=jax.ShapeDtypeStruct((S, D), jnp.float32),
        interpret=interpret,
    )(meta, W1, W3, W2, xs)


def _combine_body(slot_ref, w_ref, y_ref, out_ref):
    s = slot_ref[...]
    w = w_ref[...]
    sid = jax.lax.broadcasted_iota(jnp.int32, (TCHUNK, S), 1)
    comb = (jnp.where(sid == s[:, 0:1], w[:, 0:1], 0.0)
            + jnp.where(sid == s[:, 1:2], w[:, 1:2], 0.0))
    out_ref[...] = jax.lax.dot_general(
        comb, y_ref[...], (((1,), (0,)), ((), ())),
        preferred_element_type=jnp.float32)


def _combine_call(slot, w, y, *, interpret=False):
    return pl.pallas_call(
        _combine_body,
        grid=(NCHUNK,),
        in_specs=[
            pl.BlockSpec((TCHUNK, 2), lambda c: (c, 0)),
            pl.BlockSpec((TCHUNK, 2), lambda c: (c, 0)),
            pl.BlockSpec((S, D), lambda c: (0, 0)),
        ],
        out_specs=pl.BlockSpec((TCHUNK, D), lambda c: (c, 0)),
        out_shape=jax.ShapeDtypeStruct((T, D), jnp.float32),
        interpret=interpret,
    )(slot, w, y)


def _moe_impl(hidden_states, Wg, W1, W2, W3, *, interpret=False):
    x = hidden_states.reshape(T, D)
    e, w, r, cnt = _gate_call(x, Wg, interpret=interpret)
    counts = cnt[0].astype(jnp.int32)
    pblocks = (counts + BM - 1) // BM
    starts = jnp.concatenate(
        [jnp.zeros((1,), jnp.int32), jnp.cumsum(pblocks)[:-1]])
    off = starts * BM
    slot = jnp.take(off, e, axis=0) + r.astype(jnp.int32)
    slot0 = slot[:, 0].reshape(1, T)
    slot1 = slot[:, 1].reshape(1, T)
    meta = jnp.concatenate([pblocks, starts]).astype(jnp.int32)
    xs = _dispatch_call(slot0, slot1, x, interpret=interpret)
    y = _ffn_call(meta, xs, W1, W3, W2, interpret=interpret)
    out = _combine_call(slot, w, y, interpret=interpret)
    return out.reshape(hidden_states.shape)


def kernel(hidden_states, Wg, W1, W2, W3):
    return _moe_impl(hidden_states, Wg, W1, W2, W3)
